# Initial kernel scaffold; baseline (speedup 1.0000x reference)
import jax
import jax.numpy as jnp
from jax.experimental import pallas as pl


def kernel(x0, x1, edges, rel_dist, basis00, basis01, basis10, basis11, params, neighbor_indices, neighbor_masks):
    raise NotImplementedError("write your pallas kernel here")


# speedup vs baseline: 6.4142x; 6.4142x over previous
import functools

import jax
import jax.numpy as jnp
from jax import lax
from jax.experimental import pallas as pl
from jax.experimental.pallas import tpu as pltpu
from jax.experimental.pallas import tpu_sc as plsc

N = 10000
K = 8
D0 = 16
D1 = 4
X0W = D0
X1W = D1 * 3
TBLW = 32
NPAD = 10240
NB = NPAD // 128
EB = 128 * K
NW = 32
EPW = NPAD * K // NW
NCHUNK = EPW // 128

PAIR_OUT = {(0, 0): D0 * D0, (0, 1): D1 * D0, (1, 0): D0 * D1, (1, 1): D1 * D1 * 3}


def _sc_gather(table, idx3):
    mesh = plsc.VectorSubcoreMesh(core_axis_name="c", subcore_axis_name="s")

    @functools.partial(
        pl.kernel,
        mesh=mesh,
        out_type=jax.ShapeDtypeStruct((NPAD * K, TBLW), jnp.float32),
        scratch_types=[
            pltpu.VMEM((NCHUNK, 128), jnp.int32),
            pltpu.VMEM((EPW, TBLW), jnp.float32),
            pltpu.SemaphoreType.DMA,
        ],
        compiler_params=pltpu.CompilerParams(use_tc_tiling_on_sc=False),
    )
    def k(table_hbm, idx_hbm, out_hbm, idx_v, rows_v, sem):
        wid = lax.axis_index("s") * 2 + lax.axis_index("c")
        pltpu.sync_copy(idx_hbm.at[wid], idx_v)
        copies = []
        for j in range(NCHUNK):
            copies.append(
                pltpu.async_copy(
                    table_hbm.at[idx_v.at[j]],
                    rows_v.at[pl.ds(j * 128, 128)],
                    sem,
                )
            )
        for c in copies:
            c.wait()
        pltpu.sync_copy(rows_v, out_hbm.at[pl.ds(wid * EPW, EPW)])

    return k(table, idx3)


def _ln_t(h, g, b):
    mu = jnp.mean(h, axis=0, keepdims=True)
    var = jnp.mean((h - mu) ** 2, axis=0, keepdims=True)
    return (h - mu) * lax.rsqrt(var + 1e-5) * g + b


def _mlp_t(feat, W1, b1, g1, be1, W2, b2, g2, be2, W3, b3):
    cn = (((0,), (0,)), ((), ()))
    h = lax.dot_general(W1, feat, cn, preferred_element_type=jnp.float32)
    h = jax.nn.relu(_ln_t(h + b1, g1, be1))
    h = lax.dot_general(W2, h, cn, preferred_element_type=jnp.float32)
    h = jax.nn.relu(_ln_t(h + b2, g2, be2))
    return lax.dot_general(W3, h, cn, preferred_element_type=jnp.float32) + b3


def _contract(R, y, O, I):
    E = R.shape[-1]
    return (R.reshape(O, I, E) * y[None, :, :]).sum(axis=1)


def _tc_body(feat_r, b00_r, b01_r, b10_r, b11_r, xg_r, mask_r, x0s_r, x1s_r,
             *rest):
    (W1a, b1a, g1a, be1a, W2a, b2a, g2a, be2a, W3a, b3a,
     W1b, b1b, g1b, be1b, W2b, b2b, g2b, be2b, W3b, b3b,
     W1c, b1c, g1c, be1c, W2c, b2c, g2c, be2c, W3c, b3c,
     W1d, b1d, g1d, be1d, W2d, b2d, g2d, be2d, W3d, b3d,
     w0_r, w1_r, out0_r, out1_r) = rest

    feat = feat_r[0]
    b00 = b00_r[0]
    b01 = b01_r[0]
    b10 = b10_r[0]
    b11 = b11_r[0]
    xg = xg_r[0]
    mask = mask_r[0]
    x0s = x0s_r[0]
    x1s = x1s_r[0]

    xg0 = xg[:X0W]
    xg1 = xg[X0W:X0W + X1W]
    xg1r = xg1.reshape(D1, 3, EB)

    R00 = _mlp_t(feat, W1a[...], b1a[...], g1a[...], be1a[...], W2a[...],
                 b2a[...], g2a[...], be2a[...], W3a[...], b3a[...])
    R01 = _mlp_t(feat, W1b[...], b1b[...], g1b[...], be1b[...], W2b[...],
                 b2b[...], g2b[...], be2b[...], W3b[...], b3b[...])
    R10 = _mlp_t(feat, W1c[...], b1c[...], g1c[...], be1c[...], W2c[...],
                 b2c[...], g2c[...], be2c[...], W3c[...], b3c[...])
    R11 = _mlp_t(feat, W1d[...], b1d[...], g1d[...], be1d[...], W2d[...],
                 b2d[...], g2d[...], be2d[...], W3d[...], b3d[...])

    y0 = b00 * xg0
    acc0 = _contract(R00, y0, D0, D0)
    u1 = (xg1r * b10[None, :, :]).sum(axis=1)
    acc0 = acc0 + _contract(R10, u1, D0, D1)

    v = _contract(R01, xg0, D1, D0)
    acc1 = (v[:, None, :] * b01[None, :, :]).reshape(D1 * 3, EB)
    b11r = b11.reshape(3, 3, 3, EB)
    R11r = R11.reshape(D1, D1, 3, EB)
    parts = []
    for mo in range(3):
        part = jnp.zeros((D1, EB), jnp.float32)
        for f in range(3):
            Vmf = (xg1r * b11r[mo, :, f, :][None, :, :]).sum(axis=1)
            part = part + (R11r[:, :, f, :] * Vmf[None, :, :]).sum(axis=1)
        parts.append(part)
    accmo = jnp.stack(parts, axis=1).reshape(D1 * 3, EB)
    acc1 = acc1 + accmo

    msum = jnp.zeros((1, 128), jnp.float32)
    s0 = jnp.zeros((D0, 128), jnp.float32)
    s1 = jnp.zeros((D1 * 3, 128), jnp.float32)
    for k in range(K):
        mk = mask[k:k + 1, :]
        msum = msum + mk
        s0 = s0 + acc0[:, k * 128:(k + 1) * 128] * mk
        s1 = s1 + acc1[:, k * 128:(k + 1) * 128] * mk
    denom = jnp.maximum(msum, 1.0)
    s0 = s0 / denom
    s1 = s1 / denom

    cn = (((0,), (0,)), ((), ()))
    s0 = s0 + lax.dot_general(w0_r[...], x0s, cn,
                              preferred_element_type=jnp.float32)
    x1m = x1s.reshape(D1, 3 * 128)
    s1 = s1 + lax.dot_general(w1_r[...], x1m, cn,
                              preferred_element_type=jnp.float32).reshape(
                                  D1 * 3, 128)

    out0_r[0] = s0
    out1_r[0] = s1


def _tc_call_kwargs():
    def blk(c, e):
        return pl.BlockSpec((1, c, e), lambda i: (i, 0, 0))

    def whole(shape):
        return pl.BlockSpec(shape, lambda i: (0, 0))

    in_specs = [
        blk(5, EB), blk(1, EB), blk(3, EB), blk(3, EB), blk(27, EB),
        blk(TBLW, EB), blk(K, 128), blk(D0, 128), blk(D1 * 3, 128),
    ]
    for (di, do) in ((0, 0), (0, 1), (1, 0), (1, 1)):
        op = PAIR_OUT[(di, do)]
        in_specs += [
            whole((5, 128)), whole((128, 1)), whole((128, 1)), whole((128, 1)),
            whole((128, 128)), whole((128, 1)), whole((128, 1)), whole((128, 1)),
            whole((128, op)), whole((op, 1)),
        ]
    in_specs += [whole((D0, D0)), whole((D1, D1))]

    return dict(
        grid=(NB,),
        in_specs=in_specs,
        out_specs=[
            pl.BlockSpec((1, D0, 128), lambda i: (i, 0, 0)),
            pl.BlockSpec((1, D1 * 3, 128), lambda i: (i, 0, 0)),
        ],
        out_shape=[
            jax.ShapeDtypeStruct((NB, D0, 128), jnp.float32),
            jax.ShapeDtypeStruct((NB, D1 * 3, 128), jnp.float32),
        ],
        compiler_params=pltpu.CompilerParams(
            dimension_semantics=("arbitrary",)),
    )


def _edge_t(a, c):
    return a.reshape(NB, 128, K, c).transpose(0, 3, 2, 1).reshape(NB, c, EB)


def _node_t(a, c):
    return a.reshape(NB, 128, c).transpose(0, 2, 1)


def _padn(a):
    return jnp.pad(a, ((0, NPAD - N),) + ((0, 0),) * (a.ndim - 1))


def _flatten_params(params):
    flat = []
    for di in (0, 1):
        for do in (0, 1):
            p = params['rp%d%d' % (di, do)]
            op = PAIR_OUT[(di, do)]
            flat += [
                p['W1'], p['b1'].reshape(128, 1), p['g1'].reshape(128, 1),
                p['be1'].reshape(128, 1), p['W2'], p['b2'].reshape(128, 1),
                p['g2'].reshape(128, 1), p['be2'].reshape(128, 1),
                p['W3'], p['b3'].reshape(op, 1),
            ]
    flat += [params['w0'], params['w1']]
    return flat


def kernel(x0, x1, edges, rel_dist, basis00, basis01, basis10, basis11,
           params, neighbor_indices, neighbor_masks):
    x0f = x0.reshape(N, X0W)
    x1f = x1.reshape(N, X1W)
    table = jnp.concatenate(
        [x0f, x1f, jnp.zeros((N, TBLW - X0W - X1W), jnp.float32)], axis=1)

    idx = _padn(neighbor_indices.reshape(N, K).astype(jnp.int32))
    idx3 = idx.reshape(NW, NCHUNK, 128)

    rows = _sc_gather(table, idx3)
    xg_t = _edge_t(rows.reshape(NPAD, K, TBLW), TBLW)

    feat = jnp.concatenate([rel_dist.reshape(N, K, 1),
                            edges.reshape(N, K, 4)], axis=-1)
    feat_t = _edge_t(_padn(feat), 5)
    b00_t = _edge_t(_padn(basis00.reshape(N, K, 1)), 1)
    b01_t = _edge_t(_padn(basis01.reshape(N, K, 3)), 3)
    b10_t = _edge_t(_padn(basis10.reshape(N, K, 3)), 3)
    b11_t = _edge_t(_padn(basis11.reshape(N, K, 27)), 27)
    mask_t = _edge_t(_padn(neighbor_masks.reshape(N, K, 1)
                           .astype(jnp.float32)), 1).reshape(NB, K, 128)
    x0s_t = _node_t(_padn(x0f), X0W)
    x1s_t = _node_t(_padn(x1f), X1W)

    args = [feat_t, b00_t, b01_t, b10_t, b11_t, xg_t, mask_t, x0s_t, x1s_t]
    args += _flatten_params(params)

    out0_b, out1_b = pl.pallas_call(_tc_body, **_tc_call_kwargs())(*args)

    out0 = out0_b.transpose(0, 2, 1).reshape(NPAD, D0)[:N]
    out1 = out1_b.transpose(0, 2, 1).reshape(NPAD, D1 * 3)[:N]
    return (out0.reshape(1, N, D0, 1), out1.reshape(1, N, D1, 3))


# speedup vs baseline: 9.1311x; 1.4236x over previous
import functools

import jax
import jax.numpy as jnp
from jax import lax
from jax.experimental import pallas as pl
from jax.experimental.pallas import tpu as pltpu
from jax.experimental.pallas import tpu_sc as plsc

N = 10000
K = 8
D0 = 16
D1 = 4
X0W = D0
X1W = D1 * 3
TBLW = 32
NPAD = 10240
NBLK = 512
NB = NPAD // NBLK
EB = NBLK * K
NW = 32
EPW = NPAD * K // NW
NCHUNK = EPW // 128

PAIR_OUT = {(0, 0): D0 * D0, (0, 1): D1 * D0, (1, 0): D0 * D1, (1, 1): D1 * D1 * 3}


def _sc_gather(table, idx3):
    mesh = plsc.VectorSubcoreMesh(core_axis_name="c", subcore_axis_name="s")

    @functools.partial(
        pl.kernel,
        mesh=mesh,
        out_type=jax.ShapeDtypeStruct((NPAD * K, TBLW), jnp.float32),
        scratch_types=[
            pltpu.VMEM((NCHUNK, 128), jnp.int32),
            pltpu.VMEM((EPW, TBLW), jnp.float32),
            pltpu.SemaphoreType.DMA,
        ],
        compiler_params=pltpu.CompilerParams(use_tc_tiling_on_sc=False),
    )
    def k(table_hbm, idx_hbm, out_hbm, idx_v, rows_v, sem):
        wid = lax.axis_index("s") * 2 + lax.axis_index("c")
        pltpu.sync_copy(idx_hbm.at[wid], idx_v)
        copies = []
        for j in range(NCHUNK):
            copies.append(
                pltpu.async_copy(
                    table_hbm.at[idx_v.at[j]],
                    rows_v.at[pl.ds(j * 128, 128)],
                    sem,
                )
            )
        for c in copies:
            c.wait()
        pltpu.sync_copy(rows_v, out_hbm.at[pl.ds(wid * EPW, EPW)])

    return k(table, idx3)


_CN = (((0,), (0,)), ((), ()))


def _ln_t(h, g, b, onesc):
    mu = lax.dot_general(onesc, h, _CN, preferred_element_type=jnp.float32)
    ee = lax.dot_general(onesc, h * h, _CN, preferred_element_type=jnp.float32)
    var = ee - mu * mu
    return (h - mu) * lax.rsqrt(var + 1e-5) * g + b


def _mlp_t(feat, W1, b1, g1, be1, W2, b2, g2, be2, W3, b3, onesc):
    h = lax.dot_general(W1, feat, _CN, preferred_element_type=jnp.float32)
    h = jax.nn.relu(_ln_t(h + b1, g1, be1, onesc))
    h = lax.dot_general(W2, h, _CN, preferred_element_type=jnp.float32)
    h = jax.nn.relu(_ln_t(h + b2, g2, be2, onesc))
    return lax.dot_general(W3, h, _CN, preferred_element_type=jnp.float32) + b3


def _tc_body(feat_r, b00_r, b01_r, b10_r, b11_r, xg_r, mask_r, x0s_r, x1w_r,
             *rest):
    (W1a, b1a, g1a, be1a, W2a, b2a, g2a, be2a, W3a, b3a,
     W1b, b1b, g1b, be1b, W2b, b2b, g2b, be2b, W3b, b3b,
     W1c, b1c, g1c, be1c, W2c, b2c, g2c, be2c, W3c, b3c,
     W1d, b1d, g1d, be1d, W2d, b2d, g2d, be2d, W3d, b3d,
     w0_r, w1_r, out0_r, out1_r) = rest

    feat = feat_r[0]
    b00 = b00_r[0]
    b01 = b01_r[0]
    b10 = b10_r[0]
    b11 = b11_r[0]
    xg = xg_r[0]
    mask = mask_r[0]
    x0s = x0s_r[0]
    x1w = x1w_r[0]

    xg0 = xg[:X0W]
    xg1 = xg[X0W:X0W + X1W]

    onesc = jnp.full((128, 1), 1.0 / 128, jnp.float32)

    R00 = _mlp_t(feat, W1a[...], b1a[...], g1a[...], be1a[...], W2a[...],
                 b2a[...], g2a[...], be2a[...], W3a[...], b3a[...], onesc)
    R01 = _mlp_t(feat, W1b[...], b1b[...], g1b[...], be1b[...], W2b[...],
                 b2b[...], g2b[...], be2b[...], W3b[...], b3b[...], onesc)
    R10 = _mlp_t(feat, W1c[...], b1c[...], g1c[...], be1c[...], W2c[...],
                 b2c[...], g2c[...], be2c[...], W3c[...], b3c[...], onesc)
    R11 = _mlp_t(feat, W1d[...], b1d[...], g1d[...], be1d[...], W2d[...],
                 b2d[...], g2d[...], be2d[...], W3d[...], b3d[...], onesc)

    y0 = b00 * xg0
    acc0 = R00[0:D0] * y0[0:1]
    for i in range(1, D0):
        acc0 = acc0 + R00[i * D0:(i + 1) * D0] * y0[i:i + 1]
    u1 = (xg1[0:4] * b10[0:1] + xg1[4:8] * b10[1:2]
          + xg1[8:12] * b10[2:3])
    for i in range(D1):
        acc0 = acc0 + R10[i * D0:(i + 1) * D0] * u1[i:i + 1]

    v = R01[0:D1] * xg0[0:1]
    for i in range(1, D0):
        v = v + R01[i * D1:(i + 1) * D1] * xg0[i:i + 1]
    parts = []
    for mo in range(3):
        part = v * b01[mo:mo + 1]
        for f in range(3):
            r = mo * 9 + f
            Vmf = (xg1[0:4] * b11[r:r + 1]
                   + xg1[4:8] * b11[r + 3:r + 4]
                   + xg1[8:12] * b11[r + 6:r + 7])
            for i in range(D1):
                c = i * 12 + f * 4
                part = part + R11[c:c + 4] * Vmf[i:i + 1]
        parts.append(part)
    acc1 = jnp.concatenate(parts, axis=0)

    msum = jnp.zeros((1, NBLK), jnp.float32)
    s0 = jnp.zeros((D0, NBLK), jnp.float32)
    s1 = jnp.zeros((D1 * 3, NBLK), jnp.float32)
    for k in range(K):
        mk = mask[k:k + 1, :]
        msum = msum + mk
        s0 = s0 + acc0[:, k * NBLK:(k + 1) * NBLK] * mk
        s1 = s1 + acc1[:, k * NBLK:(k + 1) * NBLK] * mk
    denom = jnp.maximum(msum, 1.0)
    s0 = s0 / denom
    s1 = s1 / denom

    s0 = s0 + lax.dot_general(w0_r[...], x0s, _CN,
                              preferred_element_type=jnp.float32)
    si1 = lax.dot_general(w1_r[...], x1w, _CN,
                          preferred_element_type=jnp.float32)
    s1 = s1 + jnp.concatenate(
        [si1[:, m * NBLK:(m + 1) * NBLK] for m in range(3)], axis=0)

    out0_r[0] = s0
    out1_r[0] = s1


def _tc_call_kwargs():
    def blk(c, e):
        return pl.BlockSpec((1, c, e), lambda i: (i, 0, 0))

    def whole(shape):
        return pl.BlockSpec(shape, lambda i: (0, 0))

    in_specs = [
        blk(5, EB), blk(1, EB), blk(3, EB), blk(3, EB), blk(27, EB),
        blk(TBLW, EB), blk(K, NBLK), blk(D0, NBLK), blk(D1, 3 * NBLK),
    ]
    for (di, do) in ((0, 0), (0, 1), (1, 0), (1, 1)):
        op = PAIR_OUT[(di, do)]
        in_specs += [
            whole((5, 128)), whole((128, 1)), whole((128, 1)), whole((128, 1)),
            whole((128, 128)), whole((128, 1)), whole((128, 1)), whole((128, 1)),
            whole((128, op)), whole((op, 1)),
        ]
    in_specs += [whole((D0, D0)), whole((D1, D1))]

    return dict(
        grid=(NB,),
        in_specs=in_specs,
        out_specs=[
            pl.BlockSpec((1, D0, NBLK), lambda i: (i, 0, 0)),
            pl.BlockSpec((1, D1 * 3, NBLK), lambda i: (i, 0, 0)),
        ],
        out_shape=[
            jax.ShapeDtypeStruct((NB, D0, NBLK), jnp.float32),
            jax.ShapeDtypeStruct((NB, D1 * 3, NBLK), jnp.float32),
        ],
        compiler_params=pltpu.CompilerParams(
            dimension_semantics=("arbitrary",)),
    )


def _edge_t(a, c):
    return a.reshape(NB, NBLK, K, c).transpose(0, 3, 2, 1).reshape(NB, c, EB)


def _node_t(a, c):
    return a.reshape(NB, NBLK, c).transpose(0, 2, 1)


def _padn(a):
    return jnp.pad(a, ((0, NPAD - N),) + ((0, 0),) * (a.ndim - 1))


def _flatten_params(params):
    def perm3(W3, b3, shape, order):
        Wp = W3.reshape((128,) + shape).transpose((0,) + tuple(a + 1 for a in order))
        bp = b3.reshape(shape).transpose(order)
        op = W3.shape[1]
        return Wp.reshape(128, op), bp.reshape(op, 1)

    orders = {
        (0, 0): ((D0, D0), (1, 0)),
        (0, 1): ((D1, D0), (1, 0)),
        (1, 0): ((D0, D1), (1, 0)),
        (1, 1): ((D1, D1, 3), (1, 2, 0)),
    }
    flat = []
    for di in (0, 1):
        for do in (0, 1):
            p = params['rp%d%d' % (di, do)]
            shape, order = orders[(di, do)]
            W3p, b3p = perm3(p['W3'], p['b3'], shape, order)
            flat += [
                p['W1'], p['b1'].reshape(128, 1), p['g1'].reshape(128, 1),
                p['be1'].reshape(128, 1), p['W2'], p['b2'].reshape(128, 1),
                p['g2'].reshape(128, 1), p['be2'].reshape(128, 1),
                W3p, b3p,
            ]
    flat += [params['w0'], params['w1']]
    return flat


def kernel(x0, x1, edges, rel_dist, basis00, basis01, basis10, basis11,
           params, neighbor_indices, neighbor_masks):
    x0f = x0.reshape(N, X0W)
    x1mm = x1.reshape(N, D1, 3).swapaxes(1, 2).reshape(N, X1W)
    table = jnp.concatenate(
        [x0f, x1mm, jnp.zeros((N, TBLW - X0W - X1W), jnp.float32)], axis=1)

    idx = _padn(neighbor_indices.reshape(N, K).astype(jnp.int32))
    idx3 = idx.reshape(NW, NCHUNK, 128)

    rows = _sc_gather(table, idx3)
    xg_t = _edge_t(rows.reshape(NPAD, K, TBLW), TBLW)

    feat = jnp.concatenate([rel_dist.reshape(N, K, 1),
                            edges.reshape(N, K, 4)], axis=-1)
    feat_t = _edge_t(_padn(feat), 5)
    b00_t = _edge_t(_padn(basis00.reshape(N, K, 1)), 1)
    b01_t = _edge_t(_padn(basis01.reshape(N, K, 3)), 3)
    b10_t = _edge_t(_padn(basis10.reshape(N, K, 3)), 3)
    b11_t = _edge_t(_padn(basis11.reshape(N, K, 27)), 27)
    mask_t = _edge_t(_padn(neighbor_masks.reshape(N, K, 1)
                           .astype(jnp.float32)), 1).reshape(NB, K, NBLK)
    x0s_t = _node_t(_padn(x0f), X0W)
    x1w_t = (_padn(x1.reshape(N, D1, 3)).reshape(NB, NBLK, D1, 3)
             .transpose(0, 2, 3, 1).reshape(NB, D1, 3 * NBLK))

    args = [feat_t, b00_t, b01_t, b10_t, b11_t, xg_t, mask_t, x0s_t, x1w_t]
    args += _flatten_params(params)

    out0_b, out1_b = pl.pallas_call(_tc_body, **_tc_call_kwargs())(*args)

    out0 = out0_b.transpose(0, 2, 1).reshape(NPAD, D0)[:N]
    out1m = out1_b.transpose(0, 2, 1).reshape(NPAD, X1W)[:N]
    out1 = out1m.reshape(N, 3, D1).swapaxes(1, 2)
    return (out0.reshape(1, N, D0, 1), out1.reshape(1, N, D1, 3))
